# Initial kernel scaffold; baseline (speedup 1.0000x reference)
import functools

import jax
import jax.numpy as jnp
from jax import lax
from jax.experimental import pallas as pl
from jax.experimental.pallas import tpu as pltpu
from jax.experimental.pallas import tpu_sc as plsc

_NC, _NS, _L = 2, 16, 16
_NW = _NC * _NS


def _wid():
    return lax.axis_index("s") * _NC + lax.axis_index("c")


def _zero_vmem(ref, n):
    zeros = jnp.zeros((_L,), jnp.float32)

    def body(i, _):
        ref[pl.ds(i * _L, _L)] = zeros
        return 0

    lax.fori_loop(0, n // _L, body, 0)


@functools.cache
def _build(n, e):
    epw = e // _NW
    mesh = plsc.VectorSubcoreMesh(core_axis_name="c", subcore_axis_name="s",
                                  num_cores=_NC, num_subcores=_NS)

    @functools.partial(
        pl.kernel, mesh=mesh,
        out_type=jax.ShapeDtypeStruct((_NW, n), jnp.float32),
        scratch_types=[pltpu.VMEM((epw,), jnp.int32),
                       pltpu.VMEM((n,), jnp.float32)],
    )
    def hist_call(dst_hbm, out_hbm, idx_v, hist_v):
        wid = _wid()
        pltpu.sync_copy(dst_hbm.at[pl.ds(wid * epw, epw)], idx_v)
        _zero_vmem(hist_v, n)
        ones = jnp.ones((_L,), jnp.float32)

        def body(i, _):
            idx = idx_v[pl.ds(i * _L, _L)]
            plsc.addupdate_scatter(hist_v, [idx], ones)
            return 0

        lax.fori_loop(0, epw // _L, body, 0)
        pltpu.sync_copy(hist_v, out_hbm.at[wid])

    def norm_body(hist_ref, out_ref):
        deg = jnp.sum(hist_ref[...], axis=0, keepdims=True) + 1.0
        out_ref[...] = jnp.concatenate([lax.rsqrt(deg), 1.0 / deg], axis=0)

    norm_call = pl.pallas_call(
        norm_body,
        out_shape=jax.ShapeDtypeStruct((2, n), jnp.float32),
    )

    @functools.partial(
        pl.kernel, mesh=mesh,
        out_type=jax.ShapeDtypeStruct((_NW, n), jnp.float32),
        scratch_types=[pltpu.VMEM((epw,), jnp.int32),
                       pltpu.VMEM((epw,), jnp.int32),
                       pltpu.VMEM((n,), jnp.float32),
                       pltpu.VMEM((n,), jnp.float32)],
    )
    def edge_call(src_hbm, dst_hbm, dis_hbm, out_hbm, src_v, dst_v, dis_v, acc_v):
        wid = _wid()
        pltpu.sync_copy(src_hbm.at[pl.ds(wid * epw, epw)], src_v)
        pltpu.sync_copy(dst_hbm.at[pl.ds(wid * epw, epw)], dst_v)
        pltpu.sync_copy(dis_hbm, dis_v)
        _zero_vmem(acc_v, n)

        def body(i, _):
            s = src_v[pl.ds(i * _L, _L)]
            d = dst_v[pl.ds(i * _L, _L)]
            w = plsc.load_gather(dis_v, [s]) * plsc.load_gather(dis_v, [d])
            plsc.addupdate_scatter(acc_v, [s], w)
            return 0

        lax.fori_loop(0, epw // _L, body, 0)
        pltpu.sync_copy(acc_v, out_hbm.at[wid])

    def final_body(upart_ref, invdeg_ref, x_ref, w_ref, b_ref, out_ref):
        u = jnp.sum(upart_ref[...], axis=0, keepdims=True) + invdeg_ref[...]
        v = jnp.dot(u, x_ref[...], preferred_element_type=jnp.float32)
        out_ref[...] = (
            jnp.dot(v * (1.0 / n), w_ref[...], preferred_element_type=jnp.float32)
            + b_ref[...]
        )

    def final_call(upart, invdeg, x, w, b2):
        return pl.pallas_call(
            final_body,
            out_shape=jax.ShapeDtypeStruct((1, w.shape[1]), jnp.float32),
        )(upart, invdeg, x, w, b2)

    return hist_call, norm_call, edge_call, final_call


def kernel(x, edge_index, W, b):
    n = x.shape[0]
    e = edge_index.shape[1]
    hist_call, norm_call, edge_call, final_call = _build(n, e)
    ei = edge_index.astype(jnp.int32)
    src, dst = ei[0], ei[1]
    hist = hist_call(dst)
    norm2 = norm_call(hist)
    upart = edge_call(src, dst, norm2[0])
    return final_call(upart, norm2[1:2], x, W, b.reshape(1, -1))


# speedup vs baseline: 119.8486x; 119.8486x over previous
import functools

import jax
import jax.numpy as jnp
from jax import lax
from jax.experimental import pallas as pl
from jax.experimental.pallas import tpu as pltpu
from jax.experimental.pallas import tpu_sc as plsc

_NC, _NS, _L = 2, 16, 16
_NW = _NC * _NS


def _wid():
    return lax.axis_index("s") * _NC + lax.axis_index("c")


def _zero_vmem(ref, n):
    zeros = jnp.zeros((_L,), jnp.float32)

    def body(i, _):
        ref[pl.ds(i * _L, _L)] = zeros
        return 0

    lax.fori_loop(0, n // _L, body, 0)


@functools.cache
def _build(n, e):
    epw = e // _NW
    mesh = plsc.VectorSubcoreMesh(core_axis_name="c", subcore_axis_name="s",
                                  num_cores=_NC, num_subcores=_NS)
    sc_params = pltpu.CompilerParams(needs_layout_passes=False)

    @functools.partial(
        pl.kernel, mesh=mesh,
        out_type=jax.ShapeDtypeStruct((_NW, n), jnp.float32),
        scratch_types=[pltpu.VMEM((epw,), jnp.int32),
                       pltpu.VMEM((n,), jnp.float32)],
        compiler_params=sc_params,
    )
    def hist_call(dst_hbm, out_hbm, idx_v, hist_v):
        wid = _wid()
        pltpu.sync_copy(dst_hbm.at[pl.ds(wid * epw, epw)], idx_v)
        _zero_vmem(hist_v, n)
        ones = jnp.ones((_L,), jnp.float32)

        def body(i, _):
            idx = idx_v[pl.ds(i * _L, _L)]
            plsc.addupdate_scatter(hist_v, [idx], ones)
            return 0

        lax.fori_loop(0, epw // _L, body, 0)
        pltpu.sync_copy(hist_v, out_hbm.at[wid])

    def norm_body(hist_ref, out_ref):
        deg = jnp.sum(hist_ref[...], axis=0, keepdims=True) + 1.0
        out_ref[...] = jnp.concatenate([lax.rsqrt(deg), 1.0 / deg], axis=0)

    norm_call = pl.pallas_call(
        norm_body,
        out_shape=jax.ShapeDtypeStruct((2, n), jnp.float32),
    )

    @functools.partial(
        pl.kernel, mesh=mesh,
        out_type=jax.ShapeDtypeStruct((_NW, n), jnp.float32),
        scratch_types=[pltpu.VMEM((epw,), jnp.int32),
                       pltpu.VMEM((epw,), jnp.int32),
                       pltpu.VMEM((n,), jnp.float32),
                       pltpu.VMEM((n,), jnp.float32)],
        compiler_params=sc_params,
    )
    def edge_call(src_hbm, dst_hbm, dis_hbm, out_hbm, src_v, dst_v, dis_v, acc_v):
        wid = _wid()
        pltpu.sync_copy(src_hbm.at[pl.ds(wid * epw, epw)], src_v)
        pltpu.sync_copy(dst_hbm.at[pl.ds(wid * epw, epw)], dst_v)
        pltpu.sync_copy(dis_hbm, dis_v)
        _zero_vmem(acc_v, n)

        def body(i, _):
            s = src_v[pl.ds(i * _L, _L)]
            d = dst_v[pl.ds(i * _L, _L)]
            w = plsc.load_gather(dis_v, [s]) * plsc.load_gather(dis_v, [d])
            plsc.addupdate_scatter(acc_v, [s], w)
            return 0

        lax.fori_loop(0, epw // _L, body, 0)
        pltpu.sync_copy(acc_v, out_hbm.at[wid])

    def final_body(upart_ref, invdeg_ref, x_ref, w_ref, b_ref, out_ref):
        u = jnp.sum(upart_ref[...], axis=0, keepdims=True) + invdeg_ref[...]
        v = jnp.dot(u, x_ref[...], preferred_element_type=jnp.float32)
        out_ref[...] = (
            jnp.dot(v * (1.0 / n), w_ref[...], preferred_element_type=jnp.float32)
            + b_ref[...]
        )

    def final_call(upart, invdeg, x, w, b2):
        return pl.pallas_call(
            final_body,
            out_shape=jax.ShapeDtypeStruct((1, w.shape[1]), jnp.float32),
        )(upart, invdeg, x, w, b2)

    return hist_call, norm_call, edge_call, final_call


def kernel(x, edge_index, W, b):
    n = x.shape[0]
    e = edge_index.shape[1]
    hist_call, norm_call, edge_call, final_call = _build(n, e)
    ei = edge_index.astype(jnp.int32)
    src, dst = ei[0], ei[1]
    hist = hist_call(dst)
    norm2 = norm_call(hist)
    upart = edge_call(src, dst, norm2[0])
    return final_call(upart, norm2[1:2], x, W, b.reshape(1, -1))


# speedup vs baseline: 153.3205x; 1.2793x over previous
import functools

import jax
import jax.numpy as jnp
from jax import lax
from jax.experimental import pallas as pl
from jax.experimental.pallas import tpu as pltpu
from jax.experimental.pallas import tpu_sc as plsc

_NC, _NS, _L = 2, 16, 16
_NW = _NC * _NS


def _wid():
    return lax.axis_index("s") * _NC + lax.axis_index("c")


def _zero_vmem(ref, n):
    zeros = jnp.zeros((_L,), jnp.float32)

    @plsc.parallel_loop(0, n, step=_L, unroll=8)
    def _(i):
        ref[pl.ds(i, _L)] = zeros


@functools.cache
def _build(n, e):
    epw = e // _NW
    mesh = plsc.VectorSubcoreMesh(core_axis_name="c", subcore_axis_name="s",
                                  num_cores=_NC, num_subcores=_NS)
    sc_params = pltpu.CompilerParams(needs_layout_passes=False)

    @functools.partial(
        pl.kernel, mesh=mesh,
        out_type=jax.ShapeDtypeStruct((_NW, n), jnp.float32),
        scratch_types=[pltpu.VMEM((epw,), jnp.int32),
                       pltpu.VMEM((n,), jnp.float32)],
        compiler_params=sc_params,
    )
    def hist_call(dst_hbm, out_hbm, idx_v, hist_v):
        wid = _wid()
        pltpu.sync_copy(dst_hbm.at[pl.ds(wid * epw, epw)], idx_v)
        _zero_vmem(hist_v, n)
        ones = jnp.ones((_L,), jnp.float32)

        @plsc.parallel_loop(0, epw, step=_L, unroll=8)
        def _(i):
            plsc.addupdate_scatter(hist_v, [idx_v[pl.ds(i, _L)]], ones)

        pltpu.sync_copy(hist_v, out_hbm.at[wid])

    def norm_body(hist_ref, out_ref):
        deg = jnp.sum(hist_ref[...], axis=0, keepdims=True) + 1.0
        out_ref[...] = jnp.concatenate([lax.rsqrt(deg), 1.0 / deg], axis=0)

    norm_call = pl.pallas_call(
        norm_body,
        out_shape=jax.ShapeDtypeStruct((2, n), jnp.float32),
    )

    @functools.partial(
        pl.kernel, mesh=mesh,
        out_type=jax.ShapeDtypeStruct((_NW, n), jnp.float32),
        scratch_types=[pltpu.VMEM((epw,), jnp.int32),
                       pltpu.VMEM((epw,), jnp.int32),
                       pltpu.VMEM((n,), jnp.float32),
                       pltpu.VMEM((n,), jnp.float32)],
        compiler_params=sc_params,
    )
    def edge_call(src_hbm, dst_hbm, norm_hbm, out_hbm, src_v, dst_v, dis_v, acc_v):
        wid = _wid()
        pltpu.sync_copy(src_hbm.at[pl.ds(wid * epw, epw)], src_v)
        pltpu.sync_copy(dst_hbm.at[pl.ds(wid * epw, epw)], dst_v)
        pltpu.sync_copy(norm_hbm.at[0], dis_v)
        _zero_vmem(acc_v, n)

        @plsc.parallel_loop(0, epw, step=_L, unroll=8)
        def _(i):
            w = plsc.load_gather(dis_v, [dst_v[pl.ds(i, _L)]])
            plsc.addupdate_scatter(acc_v, [src_v[pl.ds(i, _L)]], w)

        pltpu.sync_copy(acc_v, out_hbm.at[wid])

    def final_body(tpart_ref, norm_ref, x_ref, w_ref, b_ref, out_ref):
        t = jnp.sum(tpart_ref[...], axis=0, keepdims=True)
        u = norm_ref[0:1, :] * t + norm_ref[1:2, :]
        v = jnp.dot(u, x_ref[...], preferred_element_type=jnp.float32)
        out_ref[...] = (
            jnp.dot(v * (1.0 / n), w_ref[...], preferred_element_type=jnp.float32)
            + b_ref[...]
        )

    def final_call(tpart, norm2, x, w, b2):
        return pl.pallas_call(
            final_body,
            out_shape=jax.ShapeDtypeStruct((1, w.shape[1]), jnp.float32),
        )(tpart, norm2, x, w, b2)

    return hist_call, norm_call, edge_call, final_call


def kernel(x, edge_index, W, b):
    n = x.shape[0]
    e = edge_index.shape[1]
    hist_call, norm_call, edge_call, final_call = _build(n, e)
    ei = edge_index.astype(jnp.int32)
    src, dst = ei[0], ei[1]
    hist = hist_call(dst)
    norm2 = norm_call(hist)
    tpart = edge_call(src, dst, norm2)
    return final_call(tpart, norm2, x, W, b.reshape(1, -1))


# speedup vs baseline: 162.9299x; 1.0627x over previous
import functools

import jax
import jax.numpy as jnp
from jax import lax
from jax.experimental import pallas as pl
from jax.experimental.pallas import tpu as pltpu
from jax.experimental.pallas import tpu_sc as plsc

_NC, _NS, _L = 2, 16, 16
_NW = _NC * _NS


def _zero_vmem(ref, n):
    zeros = jnp.zeros((_L,), jnp.float32)

    @plsc.parallel_loop(0, n, step=_L, unroll=8)
    def _(i):
        ref[pl.ds(i, _L)] = zeros


def _rsqrt_newton(d):
    half = 0.5 * d
    yi = jnp.full((_L,), 0x5F3759DF, jnp.int32) - lax.shift_right_logical(
        plsc.bitcast(d, jnp.int32), jnp.full((_L,), 1, jnp.int32))
    y = plsc.bitcast(yi, jnp.float32)
    for _ in range(3):
        y = y * (1.5 - half * y * y)
    return y


@functools.cache
def _build(n, e):
    n_pad = ((n + (_L * _NS) - 1) // (_L * _NS)) * (_L * _NS)
    cols = n_pad // _NS
    epc = e // _NS
    epw = e // _NW
    mesh = plsc.VectorSubcoreMesh(core_axis_name="c", subcore_axis_name="s",
                                  num_cores=_NC, num_subcores=_NS)
    sc_params = pltpu.CompilerParams(needs_layout_passes=False)

    @functools.partial(
        pl.kernel, mesh=mesh,
        out_type=(jax.ShapeDtypeStruct((_NW, n_pad), jnp.float32),
                  jax.ShapeDtypeStruct((n_pad,), jnp.float32),
                  jax.ShapeDtypeStruct((n_pad,), jnp.float32)),
        scratch_types=[pltpu.VMEM((epc,), jnp.int32),
                       pltpu.VMEM((epw,), jnp.int32),
                       pltpu.VMEM((n_pad,), jnp.float32),
                       pltpu.VMEM((n_pad,), jnp.float32),
                       pltpu.VMEM((_NS, cols), jnp.float32),
                       pltpu.VMEM((cols,), jnp.float32),
                       pltpu.VMEM((cols,), jnp.float32),
                       pltpu.VMEM_SHARED((_NS, n_pad), jnp.float32),
                       pltpu.VMEM_SHARED((n_pad,), jnp.float32)],
        compiler_params=sc_params,
    )
    def sc_call(src_hbm, dst_hbm, tpart_hbm, dis_hbm, inv_hbm,
                idx_v, src_v, hist_v, dis_v, blk_v, diss_v, invs_v,
                hist_sh, dis_sh):
        cid = lax.axis_index("c")
        sid = lax.axis_index("s")
        wid = sid * _NC + cid

        pltpu.sync_copy(dst_hbm.at[pl.ds(sid * epc, epc)], idx_v)
        _zero_vmem(hist_v, n_pad)
        ones = jnp.ones((_L,), jnp.float32)

        @plsc.parallel_loop(0, epc, step=_L, unroll=8)
        def _(i):
            plsc.addupdate_scatter(hist_v, [idx_v[pl.ds(i, _L)]], ones)

        pltpu.sync_copy(hist_v, hist_sh.at[sid])
        plsc.subcore_barrier()

        pltpu.sync_copy(hist_sh.at[:, pl.ds(sid * cols, cols)], blk_v)

        @plsc.parallel_loop(0, cols, step=_L, unroll=2)
        def _(j):
            acc = blk_v[0, pl.ds(j, _L)]
            for r in range(1, _NS):
                acc = acc + blk_v[r, pl.ds(j, _L)]
            d = acc + 1.0
            y = _rsqrt_newton(d)
            diss_v[pl.ds(j, _L)] = y
            invs_v[pl.ds(j, _L)] = 1.0 / d

        pltpu.sync_copy(diss_v, dis_sh.at[pl.ds(sid * cols, cols)])

        @pl.when(cid == 0)
        def _():
            pltpu.sync_copy(diss_v, dis_hbm.at[pl.ds(sid * cols, cols)])
            pltpu.sync_copy(invs_v, inv_hbm.at[pl.ds(sid * cols, cols)])

        plsc.subcore_barrier()

        pltpu.sync_copy(dis_sh, dis_v)
        pltpu.sync_copy(src_hbm.at[pl.ds(wid * epw, epw)], src_v)
        pltpu.sync_copy(dst_hbm.at[pl.ds(wid * epw, epw)], idx_v.at[pl.ds(0, epw)])
        _zero_vmem(hist_v, n_pad)

        @plsc.parallel_loop(0, epw, step=_L, unroll=8)
        def _(i):
            w = plsc.load_gather(dis_v, [idx_v[pl.ds(i, _L)]])
            plsc.addupdate_scatter(hist_v, [src_v[pl.ds(i, _L)]], w)

        pltpu.sync_copy(hist_v, tpart_hbm.at[wid])

    def final_body(tpart_ref, dis_ref, inv_ref, x_ref, w_ref, b_ref, out_ref):
        t = jnp.sum(tpart_ref[...], axis=0)
        u = (dis_ref[...] * t + inv_ref[...])[:n].reshape(1, n)
        v = jnp.dot(u, x_ref[...], preferred_element_type=jnp.float32)
        out_ref[...] = (
            jnp.dot(v * (1.0 / n), w_ref[...], preferred_element_type=jnp.float32)
            + b_ref[...]
        )

    def final_call(tpart, dis, inv, x, w, b2):
        return pl.pallas_call(
            final_body,
            out_shape=jax.ShapeDtypeStruct((1, w.shape[1]), jnp.float32),
        )(tpart, dis, inv, x, w, b2)

    return sc_call, final_call


def kernel(x, edge_index, W, b):
    n = x.shape[0]
    e = edge_index.shape[1]
    sc_call, final_call = _build(n, e)
    ei = edge_index.astype(jnp.int32)
    src, dst = ei[0], ei[1]
    tpart, dis, inv = sc_call(src, dst)
    return final_call(tpart, dis, inv, x, W, b.reshape(1, -1))


# speedup vs baseline: 211.8902x; 1.3005x over previous
import functools

import jax
import jax.numpy as jnp
from jax import lax
from jax.experimental import pallas as pl
from jax.experimental.pallas import tpu as pltpu
from jax.experimental.pallas import tpu_sc as plsc

_NC, _NS, _L = 2, 16, 16
_NW = _NC * _NS
_B = 128


def _zero_vmem(ref, n):
    zeros = jnp.zeros((_L,), jnp.float32)

    @plsc.parallel_loop(0, n, step=_L, unroll=8)
    def _(i):
        ref[pl.ds(i, _L)] = zeros


def _rsqrt_newton(d):
    half = 0.5 * d
    yi = jnp.full((_L,), 0x5F3759DF, jnp.int32) - lax.shift_right_logical(
        plsc.bitcast(d, jnp.int32), jnp.full((_L,), 1, jnp.int32))
    y = plsc.bitcast(yi, jnp.float32)
    for _ in range(3):
        y = y * (1.5 - half * y * y)
    return y


@functools.cache
def _build(n, e):
    n_pad = ((n + (_L * _NS) - 1) // (_L * _NS)) * (_L * _NS)
    cols = n_pad // _NS
    nblk = e // _B
    bpt1 = nblk // _NS
    rem1 = nblk - bpt1 * _NS
    bpt3 = nblk // _NW
    rem3 = nblk - bpt3 * _NW
    w1, w3 = bpt1 * _B, bpt3 * _B
    mesh = plsc.VectorSubcoreMesh(core_axis_name="c", subcore_axis_name="s",
                                  num_cores=_NC, num_subcores=_NS)
    sc_params = pltpu.CompilerParams(needs_layout_passes=False)

    @functools.partial(
        pl.kernel, mesh=mesh,
        out_type=(jax.ShapeDtypeStruct((_NW, n_pad), jnp.float32),
                  jax.ShapeDtypeStruct((n_pad,), jnp.float32),
                  jax.ShapeDtypeStruct((n_pad,), jnp.float32)),
        scratch_types=[pltpu.VMEM((2, w1 + _B), jnp.int32),
                       pltpu.VMEM((n_pad,), jnp.float32),
                       pltpu.VMEM((n_pad,), jnp.float32),
                       pltpu.VMEM((_NS, cols), jnp.float32),
                       pltpu.VMEM((cols,), jnp.float32),
                       pltpu.VMEM((cols,), jnp.float32),
                       pltpu.VMEM_SHARED((_NS, n_pad), jnp.float32),
                       pltpu.VMEM_SHARED((n_pad,), jnp.float32)],
        compiler_params=sc_params,
    )
    def sc_call(ei_hbm, tpart_hbm, dis_hbm, inv_hbm,
                eb_v, hist_v, dis_v, blk_v, diss_v, invs_v,
                hist_sh, dis_sh):
        cid = lax.axis_index("c")
        sid = lax.axis_index("s")
        wid = sid * _NC + cid
        ones = jnp.ones((_L,), jnp.float32)

        pltpu.sync_copy(ei_hbm.at[:, pl.ds(sid * w1, w1)],
                        eb_v.at[:, pl.ds(0, w1)])

        @pl.when(sid < rem1)
        def _():
            pltpu.sync_copy(ei_hbm.at[:, pl.ds((_NS * bpt1 + sid) * _B, _B)],
                            eb_v.at[:, pl.ds(w1, _B)])

        _zero_vmem(hist_v, n_pad)

        @plsc.parallel_loop(0, w1, step=_L, unroll=8)
        def _(i):
            plsc.addupdate_scatter(hist_v, [eb_v[1, pl.ds(i, _L)]], ones)

        @pl.when(sid < rem1)
        def _():
            @plsc.parallel_loop(0, _B, step=_L)
            def _(i):
                plsc.addupdate_scatter(hist_v, [eb_v[1, pl.ds(w1 + i, _L)]], ones)

        pltpu.sync_copy(hist_v, hist_sh.at[sid])
        plsc.subcore_barrier()

        pltpu.sync_copy(hist_sh.at[:, pl.ds(sid * cols, cols)], blk_v)

        @plsc.parallel_loop(0, cols, step=_L, unroll=2)
        def _(j):
            acc = blk_v[0, pl.ds(j, _L)]
            for r in range(1, _NS):
                acc = acc + blk_v[r, pl.ds(j, _L)]
            d = acc + 1.0
            y = _rsqrt_newton(d)
            diss_v[pl.ds(j, _L)] = y
            invs_v[pl.ds(j, _L)] = 1.0 / d

        pltpu.sync_copy(diss_v, dis_sh.at[pl.ds(sid * cols, cols)])

        @pl.when(cid == 0)
        def _():
            pltpu.sync_copy(diss_v, dis_hbm.at[pl.ds(sid * cols, cols)])
            pltpu.sync_copy(invs_v, inv_hbm.at[pl.ds(sid * cols, cols)])

        plsc.subcore_barrier()

        pltpu.sync_copy(dis_sh, dis_v)
        pltpu.sync_copy(ei_hbm.at[:, pl.ds(wid * w3, w3)],
                        eb_v.at[:, pl.ds(0, w3)])

        @pl.when(wid < rem3)
        def _():
            pltpu.sync_copy(ei_hbm.at[:, pl.ds((_NW * bpt3 + wid) * _B, _B)],
                            eb_v.at[:, pl.ds(w3, _B)])

        _zero_vmem(hist_v, n_pad)

        @plsc.parallel_loop(0, w3, step=_L, unroll=8)
        def _(i):
            w = plsc.load_gather(dis_v, [eb_v[1, pl.ds(i, _L)]])
            plsc.addupdate_scatter(hist_v, [eb_v[0, pl.ds(i, _L)]], w)

        @pl.when(wid < rem3)
        def _():
            @plsc.parallel_loop(0, _B, step=_L)
            def _(i):
                w = plsc.load_gather(dis_v, [eb_v[1, pl.ds(w3 + i, _L)]])
                plsc.addupdate_scatter(hist_v, [eb_v[0, pl.ds(w3 + i, _L)]], w)

        pltpu.sync_copy(hist_v, tpart_hbm.at[wid])

    def final_body(tpart_ref, dis_ref, inv_ref, x_ref, w_ref, b_ref, out_ref):
        t = jnp.sum(tpart_ref[...], axis=0)
        u = (dis_ref[...] * t + inv_ref[...])[:n].reshape(1, n)
        v = jnp.dot(u, x_ref[...], preferred_element_type=jnp.float32)
        out_ref[...] = (
            jnp.dot(v * (1.0 / n), w_ref[...], preferred_element_type=jnp.float32)
            + b_ref[...]
        )

    def final_call(tpart, dis, inv, x, w, b2):
        return pl.pallas_call(
            final_body,
            out_shape=jax.ShapeDtypeStruct((1, w.shape[1]), jnp.float32),
        )(tpart, dis, inv, x, w, b2)

    return sc_call, final_call


def kernel(x, edge_index, W, b):
    n = x.shape[0]
    e = edge_index.shape[1]
    sc_call, final_call = _build(n, e)
    tpart, dis, inv = sc_call(edge_index.astype(jnp.int32))
    return final_call(tpart, dis, inv, x, W, b.reshape(1, -1))


# speedup vs baseline: 223.9019x; 1.0567x over previous
import functools

import jax
import jax.numpy as jnp
from jax import lax
from jax.experimental import pallas as pl
from jax.experimental.pallas import tpu as pltpu
from jax.experimental.pallas import tpu_sc as plsc

_NC, _NS, _L = 2, 16, 16
_NW = _NC * _NS
_B = 128


def _zero_vmem(ref, n):
    zeros = jnp.zeros((_L,), jnp.float32)

    @plsc.parallel_loop(0, n, step=_L, unroll=8)
    def _(i):
        ref[pl.ds(i, _L)] = zeros


def _rsqrt_newton(d):
    half = 0.5 * d
    yi = jnp.full((_L,), 0x5F3759DF, jnp.int32) - lax.shift_right_logical(
        plsc.bitcast(d, jnp.int32), jnp.full((_L,), 1, jnp.int32))
    y = plsc.bitcast(yi, jnp.float32)
    for _ in range(3):
        y = y * (1.5 - half * y * y)
    return y


@functools.cache
def _build(n, e):
    n_pad = ((n + (_L * _NS) - 1) // (_L * _NS)) * (_L * _NS)
    cols = n_pad // _NS
    nblk = e // _B
    bpt1 = nblk // _NS
    rem1 = nblk - bpt1 * _NS
    w1 = bpt1 * _B
    half = w1 // 2
    mesh = plsc.VectorSubcoreMesh(core_axis_name="c", subcore_axis_name="s",
                                  num_cores=_NC, num_subcores=_NS)
    sc_params = pltpu.CompilerParams(needs_layout_passes=False)

    @functools.partial(
        pl.kernel, mesh=mesh,
        out_type=(jax.ShapeDtypeStruct((_NW, n_pad), jnp.float32),
                  jax.ShapeDtypeStruct((n_pad,), jnp.float32),
                  jax.ShapeDtypeStruct((n_pad,), jnp.float32)),
        scratch_types=[pltpu.VMEM((2, w1 + _B), jnp.int32),
                       pltpu.VMEM((n_pad,), jnp.float32),
                       pltpu.VMEM((n_pad,), jnp.float32),
                       pltpu.VMEM((_NS, cols), jnp.float32),
                       pltpu.VMEM((cols,), jnp.float32),
                       pltpu.VMEM((cols,), jnp.float32),
                       pltpu.VMEM_SHARED((_NS, n_pad), jnp.float32),
                       pltpu.VMEM_SHARED((n_pad,), jnp.float32)],
        compiler_params=sc_params,
    )
    def sc_call(ei_hbm, tpart_hbm, dis_hbm, inv_hbm,
                eb_v, hist_v, dis_v, blk_v, diss_v, invs_v,
                hist_sh, dis_sh):
        cid = lax.axis_index("c")
        sid = lax.axis_index("s")
        wid = sid * _NC + cid
        ones = jnp.ones((_L,), jnp.float32)

        pltpu.sync_copy(ei_hbm.at[:, pl.ds(sid * w1, w1)],
                        eb_v.at[:, pl.ds(0, w1)])

        @pl.when(sid < rem1)
        def _():
            pltpu.sync_copy(ei_hbm.at[:, pl.ds((_NS * bpt1 + sid) * _B, _B)],
                            eb_v.at[:, pl.ds(w1, _B)])

        _zero_vmem(hist_v, n_pad)

        @plsc.parallel_loop(0, w1, step=_L, unroll=8)
        def _(i):
            plsc.addupdate_scatter(hist_v, [eb_v[1, pl.ds(i, _L)]], ones)

        @pl.when(sid < rem1)
        def _():
            @plsc.parallel_loop(0, _B, step=_L)
            def _(i):
                plsc.addupdate_scatter(hist_v, [eb_v[1, pl.ds(w1 + i, _L)]], ones)

        pltpu.sync_copy(hist_v, hist_sh.at[sid])
        plsc.subcore_barrier()

        pltpu.sync_copy(hist_sh.at[:, pl.ds(sid * cols, cols)], blk_v)

        @plsc.parallel_loop(0, cols, step=_L, unroll=2)
        def _(j):
            acc = blk_v[0, pl.ds(j, _L)]
            for r in range(1, _NS):
                acc = acc + blk_v[r, pl.ds(j, _L)]
            d = acc + 1.0
            y = _rsqrt_newton(d)
            diss_v[pl.ds(j, _L)] = y
            invs_v[pl.ds(j, _L)] = 1.0 / d

        pltpu.sync_copy(diss_v, dis_sh.at[pl.ds(sid * cols, cols)])

        @pl.when(cid == 0)
        def _():
            pltpu.sync_copy(diss_v, dis_hbm.at[pl.ds(sid * cols, cols)])
            pltpu.sync_copy(invs_v, inv_hbm.at[pl.ds(sid * cols, cols)])

        plsc.subcore_barrier()

        pltpu.sync_copy(dis_sh, dis_v)
        _zero_vmem(hist_v, n_pad)
        base3 = cid * half

        @plsc.parallel_loop(0, half, step=_L, unroll=8)
        def _(i):
            w = plsc.load_gather(dis_v, [eb_v[1, pl.ds(base3 + i, _L)]])
            plsc.addupdate_scatter(hist_v, [eb_v[0, pl.ds(base3 + i, _L)]], w)

        @pl.when((sid < rem1) & (cid == sid % _NC))
        def _():
            @plsc.parallel_loop(0, _B, step=_L)
            def _(i):
                w = plsc.load_gather(dis_v, [eb_v[1, pl.ds(w1 + i, _L)]])
                plsc.addupdate_scatter(hist_v, [eb_v[0, pl.ds(w1 + i, _L)]], w)

        pltpu.sync_copy(hist_v, tpart_hbm.at[wid])

    def final_body(tpart_ref, dis_ref, inv_ref, x_ref, w_ref, b_ref, out_ref):
        t = jnp.sum(tpart_ref[...], axis=0)
        u = (dis_ref[...] * t + inv_ref[...])[:n].reshape(1, n)
        v = jnp.dot(u, x_ref[...], preferred_element_type=jnp.float32)
        out_ref[...] = (
            jnp.dot(v * (1.0 / n), w_ref[...], preferred_element_type=jnp.float32)
            + b_ref[...]
        )

    def final_call(tpart, dis, inv, x, w, b2):
        return pl.pallas_call(
            final_body,
            out_shape=jax.ShapeDtypeStruct((1, w.shape[1]), jnp.float32),
        )(tpart, dis, inv, x, w, b2)

    return sc_call, final_call


def kernel(x, edge_index, W, b):
    n = x.shape[0]
    e = edge_index.shape[1]
    sc_call, final_call = _build(n, e)
    tpart, dis, inv = sc_call(edge_index.astype(jnp.int32))
    return final_call(tpart, dis, inv, x, W, b.reshape(1, -1))


# speedup vs baseline: 233.7093x; 1.0438x over previous
import functools

import jax
import jax.numpy as jnp
from jax import lax
from jax.experimental import pallas as pl
from jax.experimental.pallas import tpu as pltpu
from jax.experimental.pallas import tpu_sc as plsc

_NC, _NS, _L = 2, 16, 16
_NW = _NC * _NS
_B = 128


def _zero_vmem(ref, n):
    zeros = jnp.zeros((_L,), jnp.float32)

    @plsc.parallel_loop(0, n, step=_L, unroll=8)
    def _(i):
        ref[pl.ds(i, _L)] = zeros


def _rsqrt_newton(d):
    half = 0.5 * d
    yi = jnp.full((_L,), 0x5F3759DF, jnp.int32) - lax.shift_right_logical(
        plsc.bitcast(d, jnp.int32), jnp.full((_L,), 1, jnp.int32))
    y = plsc.bitcast(yi, jnp.float32)
    for _ in range(3):
        y = y * (1.5 - half * y * y)
    return y


@functools.cache
def _build(n, e):
    n_pad = ((n + (_L * _NS) - 1) // (_L * _NS)) * (_L * _NS)
    cols = n_pad // _NS
    nblk = e // _B
    bpt1 = nblk // _NS
    rem1 = nblk - bpt1 * _NS
    w1 = bpt1 * _B
    half = w1 // 2
    mesh = plsc.VectorSubcoreMesh(core_axis_name="c", subcore_axis_name="s",
                                  num_cores=_NC, num_subcores=_NS)
    sc_params = pltpu.CompilerParams(needs_layout_passes=False)
    nch = 4
    csz = w1 // nch

    @functools.partial(
        pl.kernel, mesh=mesh,
        out_type=(jax.ShapeDtypeStruct((_NW, n_pad), jnp.float32),
                  jax.ShapeDtypeStruct((n_pad,), jnp.float32),
                  jax.ShapeDtypeStruct((n_pad,), jnp.float32)),
        scratch_types=[pltpu.VMEM((2, w1 + _B), jnp.int32),
                       pltpu.VMEM((n_pad,), jnp.float32),
                       pltpu.VMEM((n_pad,), jnp.float32),
                       pltpu.VMEM((_NS, cols), jnp.float32),
                       pltpu.VMEM((cols,), jnp.float32),
                       pltpu.VMEM((cols,), jnp.float32),
                       pltpu.VMEM_SHARED((_NS, n_pad), jnp.float32),
                       pltpu.VMEM_SHARED((n_pad,), jnp.float32),
                       pltpu.SemaphoreType.DMA,
                       pltpu.SemaphoreType.DMA],
        compiler_params=sc_params,
    )
    def sc_call(ei_hbm, tpart_hbm, dis_hbm, inv_hbm,
                eb_v, hist_v, dis_v, blk_v, diss_v, invs_v,
                hist_sh, dis_sh, sem_a, sem_b):
        cid = lax.axis_index("c")
        sid = lax.axis_index("s")
        wid = sid * _NC + cid
        ones = jnp.ones((_L,), jnp.float32)
        sems = [sem_a, sem_b]

        def chunk_copy(k, sem):
            return pltpu.async_copy(
                ei_hbm.at[:, pl.ds(sid * w1 + k * csz, csz)],
                eb_v.at[:, pl.ds(k * csz, csz)], sem)

        cps = [chunk_copy(0, sems[0])]
        _zero_vmem(hist_v, n_pad)

        @pl.when(sid < rem1)
        def _():
            pltpu.sync_copy(ei_hbm.at[:, pl.ds((_NS * bpt1 + sid) * _B, _B)],
                            eb_v.at[:, pl.ds(w1, _B)])

        for k in range(nch):
            if k + 1 < nch:
                cps.append(chunk_copy(k + 1, sems[(k + 1) % 2]))
            cps[k].wait()

            @plsc.parallel_loop(k * csz, (k + 1) * csz, step=_L, unroll=8)
            def _(i):
                plsc.addupdate_scatter(hist_v, [eb_v[1, pl.ds(i, _L)]], ones)

        @pl.when(sid < rem1)
        def _():
            @plsc.parallel_loop(0, _B, step=_L)
            def _(i):
                plsc.addupdate_scatter(hist_v, [eb_v[1, pl.ds(w1 + i, _L)]], ones)

        pltpu.sync_copy(hist_v, hist_sh.at[sid])
        plsc.subcore_barrier()

        pltpu.sync_copy(hist_sh.at[:, pl.ds(sid * cols, cols)], blk_v)

        @plsc.parallel_loop(0, cols, step=_L, unroll=2)
        def _(j):
            acc = blk_v[0, pl.ds(j, _L)]
            for r in range(1, _NS):
                acc = acc + blk_v[r, pl.ds(j, _L)]
            d = acc + 1.0
            y = _rsqrt_newton(d)
            diss_v[pl.ds(j, _L)] = y
            invs_v[pl.ds(j, _L)] = 1.0 / d

        pltpu.sync_copy(diss_v, dis_sh.at[pl.ds(sid * cols, cols)])

        @pl.when(cid == 0)
        def _():
            pltpu.sync_copy(diss_v, dis_hbm.at[pl.ds(sid * cols, cols)])
            pltpu.sync_copy(invs_v, inv_hbm.at[pl.ds(sid * cols, cols)])

        plsc.subcore_barrier()

        dis_cp = pltpu.async_copy(dis_sh, dis_v, sem_a)
        _zero_vmem(hist_v, n_pad)
        dis_cp.wait()
        base3 = cid * half

        @plsc.parallel_loop(0, half, step=_L, unroll=8)
        def _(i):
            w = plsc.load_gather(dis_v, [eb_v[1, pl.ds(base3 + i, _L)]])
            plsc.addupdate_scatter(hist_v, [eb_v[0, pl.ds(base3 + i, _L)]], w)

        @pl.when((sid < rem1) & (cid == sid % _NC))
        def _():
            @plsc.parallel_loop(0, _B, step=_L)
            def _(i):
                w = plsc.load_gather(dis_v, [eb_v[1, pl.ds(w1 + i, _L)]])
                plsc.addupdate_scatter(hist_v, [eb_v[0, pl.ds(w1 + i, _L)]], w)

        pltpu.sync_copy(hist_v, tpart_hbm.at[wid])

    def final_body(tpart_ref, dis_ref, inv_ref, x_ref, w_ref, b_ref, out_ref):
        t = jnp.sum(tpart_ref[...], axis=0)
        u = (dis_ref[...] * t + inv_ref[...])[:n].reshape(1, n)
        v = jnp.dot(u, x_ref[...], preferred_element_type=jnp.float32)
        out_ref[...] = (
            jnp.dot(v * (1.0 / n), w_ref[...], preferred_element_type=jnp.float32)
            + b_ref[...]
        )

    def final_call(tpart, dis, inv, x, w, b2):
        return pl.pallas_call(
            final_body,
            out_shape=jax.ShapeDtypeStruct((1, w.shape[1]), jnp.float32),
        )(tpart, dis, inv, x, w, b2)

    return sc_call, final_call


def kernel(x, edge_index, W, b):
    n = x.shape[0]
    e = edge_index.shape[1]
    sc_call, final_call = _build(n, e)
    tpart, dis, inv = sc_call(edge_index.astype(jnp.int32))
    return final_call(tpart, dis, inv, x, W, b.reshape(1, -1))
